# Initial kernel scaffold; baseline (speedup 1.0000x reference)
import jax
import jax.numpy as jnp
from jax.experimental import pallas as pl


def kernel(x, edge_index, edge_weight, W1, b1, W2, b2, W3, b3, W4, b4, Wres, bres):
    raise NotImplementedError("write your pallas kernel here")


# speedup vs baseline: 37.4081x; 37.4081x over previous
import jax
import jax.numpy as jnp
from jax import lax
from jax.experimental import pallas as pl
from jax.experimental.pallas import tpu as pltpu
from jax.experimental.pallas import tpu_sc as plsc

N = 10000
DIN = 128
H = 16

NC = 2
NS = 16
L = 16
NW = NC * NS
K = 128
NWIN = 80
T_EDGES = K * NWIN
E_CAP = NW * T_EDGES
NPAD = 10240
RPT = NPAD // NS

def _deg_body(dst_hbm, w_hbm, zeros_hbm, out_hbm, dst_v, w_v, deg_sh):
    cid = lax.axis_index("c")
    sid = lax.axis_index("s")
    wid = sid * NC + cid
    pltpu.sync_copy(dst_hbm.at[wid], dst_v)
    pltpu.sync_copy(w_hbm.at[wid], w_v)
    pltpu.sync_copy(zeros_hbm.at[pl.ds(sid * RPT, RPT)],
                    deg_sh.at[pl.ds(sid * RPT, RPT)])
    plsc.subcore_barrier()

    def body(g, carry):
        pltpu.sync_copy(w_v.at[g], deg_sh.at[dst_v.at[g]], add=True)
        return carry

    lax.fori_loop(0, NWIN, body, 0)
    plsc.subcore_barrier()
    pltpu.sync_copy(deg_sh.at[pl.ds(sid * RPT, RPT)],
                    out_hbm.at[cid, pl.ds(sid * RPT, RPT)])


import functools


@functools.cache
def _sc_kernels():
    mesh = plsc.VectorSubcoreMesh(
        core_axis_name="c", subcore_axis_name="s",
        num_cores=NC, num_subcores=NS,
    )
    deg_kernel = pl.kernel(
        _deg_body,
        out_type=jax.ShapeDtypeStruct((NC, NPAD), jnp.float32),
        mesh=mesh,
        scratch_types=[
            pltpu.VMEM((NWIN, K), jnp.int32),
            pltpu.VMEM((NWIN, K), jnp.float32),
            pltpu.VMEM_SHARED((NPAD,), jnp.float32),
        ],
        compiler_params=pltpu.CompilerParams(use_tc_tiling_on_sc=False),
    )
    prop_kernel = pl.kernel(
        _prop_body,
        out_type=jax.ShapeDtypeStruct((NC, NPAD, H), jnp.float32),
        mesh=mesh,
        scratch_types=[
            pltpu.VMEM((NWIN + 2, K), jnp.int32),
            pltpu.VMEM((NWIN, K), jnp.int32),
            pltpu.VMEM((NWIN, K), jnp.float32),
            pltpu.VMEM((K, H), jnp.float32),
            pltpu.VMEM((K, H), jnp.float32),
            pltpu.SemaphoreType.DMA,
            pltpu.SemaphoreType.DMA,
            pltpu.VMEM_SHARED((NPAD, H), jnp.float32),
        ],
        compiler_params=pltpu.CompilerParams(use_tc_tiling_on_sc=False),
    )
    return deg_kernel, prop_kernel


def _prop_body(t_hbm, src_hbm, dst_hbm, w_hbm, zeros_hbm, out_hbm,
               src_v, dst_v, w_v, rows_a, rows_b, sem_a, sem_b, acc_sh):
    cid = lax.axis_index("c")
    sid = lax.axis_index("s")
    wid = sid * NC + cid
    pltpu.sync_copy(src_hbm.at[wid], src_v)
    pltpu.sync_copy(dst_hbm.at[wid], dst_v)
    pltpu.sync_copy(w_hbm.at[wid], w_v)
    pltpu.sync_copy(zeros_hbm.at[pl.ds(sid * RPT, RPT)],
                    acc_sh.at[pl.ds(sid * RPT, RPT)])
    plsc.subcore_barrier()

    pltpu.async_copy(t_hbm.at[src_v.at[0]], rows_a, sem_a)
    pltpu.async_copy(t_hbm.at[src_v.at[1]], rows_b, sem_b)

    def body(g2, carry):
        for b in range(2):
            rows, sem = ((rows_a, sem_a), (rows_b, sem_b))[b]
            g = g2 * 2 + b
            pltpu.make_async_copy(t_hbm.at[src_v.at[g]], rows, sem).wait()
            for j16 in range(K // L):
                w16 = w_v[g, j16 * L:(j16 + 1) * L]
                for j in range(L):
                    e = j16 * L + j
                    ws = jnp.take_along_axis(
                        w16, jnp.full((L,), j, jnp.int32), axis=0)
                    rows[e, :] = rows[e, :] * ws
            pltpu.sync_copy(rows, acc_sh.at[dst_v.at[g]], add=True)
            pltpu.async_copy(t_hbm.at[src_v.at[g + 2]], rows, sem)
        return carry

    lax.fori_loop(0, NWIN // 2, body, 0)
    pltpu.make_async_copy(t_hbm.at[src_v.at[0]], rows_a, sem_a).wait()
    pltpu.make_async_copy(t_hbm.at[src_v.at[1]], rows_b, sem_b).wait()
    plsc.subcore_barrier()
    pltpu.sync_copy(acc_sh.at[pl.ds(sid * RPT, RPT)],
                    out_hbm.at[cid, pl.ds(sid * RPT, RPT)])


BN = 1000


def _tc_a_body(deg_ref, x_ref, W1_ref, Wres_ref, bres_ref,
               t1_ref, xres_ref, dis_ref):
    deg = deg_ref[0] + deg_ref[1] + 1.0
    dis = lax.rsqrt(deg)
    xw = jnp.dot(x_ref[...], W1_ref[...], preferred_element_type=jnp.float32)
    t1_ref[...] = xw * dis
    xres_ref[...] = (
        jnp.dot(x_ref[...], Wres_ref[...], preferred_element_type=jnp.float32)
        + bres_ref[...]
    )
    dis_ref[...] = dis


_tc_a = pl.pallas_call(
    _tc_a_body,
    grid=(N // BN,),
    in_specs=[
        pl.BlockSpec((NC, BN, 1), lambda i: (0, i, 0)),
        pl.BlockSpec((BN, DIN), lambda i: (i, 0)),
        pl.BlockSpec((DIN, H), lambda i: (0, 0)),
        pl.BlockSpec((DIN, H), lambda i: (0, 0)),
        pl.BlockSpec((1, H), lambda i: (0, 0)),
    ],
    out_specs=[
        pl.BlockSpec((BN, H), lambda i: (i, 0)),
        pl.BlockSpec((BN, H), lambda i: (i, 0)),
        pl.BlockSpec((BN, 1), lambda i: (i, 0)),
    ],
    out_shape=[
        jax.ShapeDtypeStruct((N, H), jnp.float32),
        jax.ShapeDtypeStruct((N, H), jnp.float32),
        jax.ShapeDtypeStruct((N, 1), jnp.float32),
    ],
)


def _tc_b_body(acc_ref, t_ref, res_ref, dis_ref, b_ref, Wn_ref,
               h_ref, tn_ref):
    conv = (acc_ref[0] + acc_ref[1] + t_ref[...]) * dis_ref[...] + b_ref[...]
    h = jnp.maximum(conv + res_ref[...], 0.0)
    h_ref[...] = h
    tn_ref[...] = (
        jnp.dot(h, Wn_ref[...], preferred_element_type=jnp.float32)
        * dis_ref[...]
    )


_tc_b = pl.pallas_call(
    _tc_b_body,
    grid=(N // BN,),
    in_specs=[
        pl.BlockSpec((NC, BN, H), lambda i: (0, i, 0)),
        pl.BlockSpec((BN, H), lambda i: (i, 0)),
        pl.BlockSpec((BN, H), lambda i: (i, 0)),
        pl.BlockSpec((BN, 1), lambda i: (i, 0)),
        pl.BlockSpec((1, H), lambda i: (0, 0)),
        pl.BlockSpec((H, H), lambda i: (0, 0)),
    ],
    out_specs=[
        pl.BlockSpec((BN, H), lambda i: (i, 0)),
        pl.BlockSpec((BN, H), lambda i: (i, 0)),
    ],
    out_shape=[
        jax.ShapeDtypeStruct((N, H), jnp.float32),
        jax.ShapeDtypeStruct((N, H), jnp.float32),
    ],
)


def _tc_b3_body(acc_ref, t_ref, res_ref, dis_ref, b_ref, t4_ref):
    conv = (acc_ref[0] + acc_ref[1] + t_ref[...]) * dis_ref[...] + b_ref[...]
    h = jnp.maximum(conv + res_ref[...], 0.0)
    t4_ref[...] = h * dis_ref[...]


_tc_b3 = pl.pallas_call(
    _tc_b3_body,
    grid=(N // BN,),
    in_specs=[
        pl.BlockSpec((NC, BN, H), lambda i: (0, i, 0)),
        pl.BlockSpec((BN, H), lambda i: (i, 0)),
        pl.BlockSpec((BN, H), lambda i: (i, 0)),
        pl.BlockSpec((BN, 1), lambda i: (i, 0)),
        pl.BlockSpec((1, H), lambda i: (0, 0)),
    ],
    out_specs=[pl.BlockSpec((BN, H), lambda i: (i, 0))],
    out_shape=[jax.ShapeDtypeStruct((N, H), jnp.float32)],
)


def _tc_c_body(acc_ref, t_ref, dis_ref, W4_ref, b4_ref, out_ref):
    z = (acc_ref[0] + acc_ref[1] + t_ref[...]) * dis_ref[...]
    out_ref[...] = (
        jnp.dot(z, W4_ref[...], preferred_element_type=jnp.float32)
        + b4_ref[...]
    )


_tc_c = pl.pallas_call(
    _tc_c_body,
    grid=(N // BN,),
    in_specs=[
        pl.BlockSpec((NC, BN, H), lambda i: (0, i, 0)),
        pl.BlockSpec((BN, H), lambda i: (i, 0)),
        pl.BlockSpec((BN, 1), lambda i: (i, 0)),
        pl.BlockSpec((H, 1), lambda i: (0, 0)),
        pl.BlockSpec((1, 1), lambda i: (0, 0)),
    ],
    out_specs=[pl.BlockSpec((BN, 1), lambda i: (i, 0))],
    out_shape=[jax.ShapeDtypeStruct((N, 1), jnp.float32)],
)


def kernel(x, edge_index, edge_weight, W1, b1, W2, b2, W3, b3, W4, b4,
           Wres, bres):
    src = edge_index[0].astype(jnp.int32)
    dst = edge_index[1].astype(jnp.int32)
    w = edge_weight.astype(jnp.float32)
    e_in = src.shape[0]
    pad = E_CAP - e_in

    spread = (jnp.arange(pad, dtype=jnp.int32) * 97) % N
    src_p = jnp.concatenate([src, spread]).reshape(NW, NWIN, K)
    dummy = jnp.broadcast_to(
        ((jnp.arange(2 * K, dtype=jnp.int32) * 53) % N).reshape(1, 2, K),
        (NW, 2, K),
    )
    src3 = jnp.concatenate([src_p, dummy], axis=1)
    dst3 = jnp.concatenate([dst, spread]).reshape(NW, NWIN, K)
    w3 = jnp.concatenate([w, jnp.zeros((pad,), jnp.float32)]).reshape(
        NW, NWIN, K)
    zeros_n = jnp.zeros((NPAD,), jnp.float32)
    zeros_nh = jnp.zeros((NPAD, H), jnp.float32)

    _deg_kernel, _prop_kernel = _sc_kernels()
    deg_parts = _deg_kernel(dst3, w3, zeros_n)
    t1, xres, dis = _tc_a(deg_parts.reshape(NC, NPAD, 1), x, W1, Wres,
                          bres.reshape(1, H))
    acc1 = _prop_kernel(t1, src3, dst3, w3, zeros_nh)
    h1, t2 = _tc_b(acc1, t1, xres, dis, b1.reshape(1, H), W2)
    acc2 = _prop_kernel(t2, src3, dst3, w3, zeros_nh)
    h2, t3 = _tc_b(acc2, t2, h1, dis, b2.reshape(1, H), W3)
    acc3 = _prop_kernel(t3, src3, dst3, w3, zeros_nh)
    (t4,) = _tc_b3(acc3, t3, h2, dis, b3.reshape(1, H))
    acc4 = _prop_kernel(t4, src3, dst3, w3, zeros_nh)
    (out,) = _tc_c(acc4, t4, dis, W4, b4.reshape(1, 1))
    return out


# speedup vs baseline: 43.5084x; 1.1631x over previous
import jax
import jax.numpy as jnp
from jax import lax
from jax.experimental import pallas as pl
from jax.experimental.pallas import tpu as pltpu
from jax.experimental.pallas import tpu_sc as plsc

N = 10000
DIN = 128
H = 16

NC = 2
NS = 16
L = 16
NW = NC * NS
K = 128
NWIN = 80
T_EDGES = K * NWIN
E_CAP = NW * T_EDGES
NPAD = 10240
RPT = NPAD // NS

def _deg_body(dst_hbm, w_hbm, zeros_hbm, out_hbm, dst_v, w_v, deg_sh):
    cid = lax.axis_index("c")
    sid = lax.axis_index("s")
    wid = sid * NC + cid
    pltpu.sync_copy(dst_hbm.at[wid], dst_v)
    pltpu.sync_copy(w_hbm.at[wid], w_v)
    pltpu.sync_copy(zeros_hbm.at[pl.ds(sid * RPT, RPT)],
                    deg_sh.at[pl.ds(sid * RPT, RPT)])
    plsc.subcore_barrier()

    def body(g, carry):
        pltpu.sync_copy(w_v.at[g], deg_sh.at[dst_v.at[g]], add=True)
        return carry

    lax.fori_loop(0, NWIN, body, 0)
    plsc.subcore_barrier()
    pltpu.sync_copy(deg_sh.at[pl.ds(sid * RPT, RPT)],
                    out_hbm.at[cid, pl.ds(sid * RPT, RPT)])


import functools


@functools.cache
def _sc_kernels():
    mesh = plsc.VectorSubcoreMesh(
        core_axis_name="c", subcore_axis_name="s",
        num_cores=NC, num_subcores=NS,
    )
    deg_kernel = pl.kernel(
        _deg_body,
        out_type=jax.ShapeDtypeStruct((NC, NPAD), jnp.float32),
        mesh=mesh,
        scratch_types=[
            pltpu.VMEM((NWIN, K), jnp.int32),
            pltpu.VMEM((NWIN, K), jnp.float32),
            pltpu.VMEM_SHARED((NPAD,), jnp.float32),
        ],
        compiler_params=pltpu.CompilerParams(use_tc_tiling_on_sc=False),
    )
    prop_kernel = pl.kernel(
        _prop_body,
        out_type=jax.ShapeDtypeStruct((NC, NPAD, H), jnp.float32),
        mesh=mesh,
        scratch_types=[
            pltpu.VMEM((NWIN + NBUF, K), jnp.int32),
            pltpu.VMEM((NWIN, K), jnp.int32),
            pltpu.VMEM((NWIN, K), jnp.float32),
            pltpu.VMEM((NBUF, K, H), jnp.float32),
            pltpu.SemaphoreType.DMA((NBUF,)),
            pltpu.SemaphoreType.DMA((NBUF,)),
            pltpu.VMEM_SHARED((NPAD, H), jnp.float32),
        ],
        compiler_params=pltpu.CompilerParams(use_tc_tiling_on_sc=False),
    )
    return deg_kernel, prop_kernel


NBUF = 4


def _prop_body(t_hbm, src_hbm, dst_hbm, w_hbm, zeros_hbm, out_hbm,
               src_v, dst_v, w_v, rows_v, gsems, ssems, acc_sh):
    cid = lax.axis_index("c")
    sid = lax.axis_index("s")
    wid = sid * NC + cid
    pltpu.sync_copy(src_hbm.at[wid], src_v)
    pltpu.sync_copy(dst_hbm.at[wid], dst_v)
    pltpu.sync_copy(w_hbm.at[wid], w_v)
    pltpu.sync_copy(zeros_hbm.at[pl.ds(sid * RPT, RPT)],
                    acc_sh.at[pl.ds(sid * RPT, RPT)])
    plsc.subcore_barrier()

    for b in range(NBUF):
        pltpu.async_copy(t_hbm.at[src_v.at[b]], rows_v.at[b], gsems.at[b])

    def scale(g, b):
        rows = rows_v.at[b]
        for j16 in range(K // L):
            w16 = w_v[g, j16 * L:(j16 + 1) * L]
            for j in range(L):
                e = j16 * L + j
                ws = jnp.take_along_axis(
                    w16, jnp.full((L,), j, jnp.int32), axis=0)
                rows[e, :] = rows[e, :] * ws

    def body(g4, carry):
        for b in range(NBUF):
            g = g4 * NBUF + b
            pltpu.make_async_copy(
                t_hbm.at[src_v.at[g]], rows_v.at[b], gsems.at[b]).wait()
            scale(g, b)
            pltpu.async_copy(rows_v.at[b], acc_sh.at[dst_v.at[g]],
                             ssems.at[b], add=True)
        for b in range(NBUF):
            g = g4 * NBUF + b
            pltpu.make_async_copy(rows_v.at[b], acc_sh.at[dst_v.at[g]],
                                  ssems.at[b]).wait()
            pltpu.async_copy(t_hbm.at[src_v.at[g + NBUF]], rows_v.at[b],
                             gsems.at[b])
        return carry

    lax.fori_loop(0, NWIN // NBUF, body, 0)
    for b in range(NBUF):
        pltpu.make_async_copy(
            t_hbm.at[src_v.at[b]], rows_v.at[b], gsems.at[b]).wait()
    plsc.subcore_barrier()
    pltpu.sync_copy(acc_sh.at[pl.ds(sid * RPT, RPT)],
                    out_hbm.at[cid, pl.ds(sid * RPT, RPT)])


BN = 1000


def _tc_a_body(deg_ref, x_ref, W1_ref, Wres_ref, bres_ref,
               t1_ref, xres_ref, dis_ref):
    deg = deg_ref[0] + deg_ref[1] + 1.0
    dis = lax.rsqrt(deg)
    xw = jnp.dot(x_ref[...], W1_ref[...], preferred_element_type=jnp.float32)
    t1_ref[...] = xw * dis
    xres_ref[...] = (
        jnp.dot(x_ref[...], Wres_ref[...], preferred_element_type=jnp.float32)
        + bres_ref[...]
    )
    dis_ref[...] = dis


_tc_a = pl.pallas_call(
    _tc_a_body,
    grid=(N // BN,),
    in_specs=[
        pl.BlockSpec((NC, BN, 1), lambda i: (0, i, 0)),
        pl.BlockSpec((BN, DIN), lambda i: (i, 0)),
        pl.BlockSpec((DIN, H), lambda i: (0, 0)),
        pl.BlockSpec((DIN, H), lambda i: (0, 0)),
        pl.BlockSpec((1, H), lambda i: (0, 0)),
    ],
    out_specs=[
        pl.BlockSpec((BN, H), lambda i: (i, 0)),
        pl.BlockSpec((BN, H), lambda i: (i, 0)),
        pl.BlockSpec((BN, 1), lambda i: (i, 0)),
    ],
    out_shape=[
        jax.ShapeDtypeStruct((N, H), jnp.float32),
        jax.ShapeDtypeStruct((N, H), jnp.float32),
        jax.ShapeDtypeStruct((N, 1), jnp.float32),
    ],
)


def _tc_b_body(acc_ref, t_ref, res_ref, dis_ref, b_ref, Wn_ref,
               h_ref, tn_ref):
    conv = (acc_ref[0] + acc_ref[1] + t_ref[...]) * dis_ref[...] + b_ref[...]
    h = jnp.maximum(conv + res_ref[...], 0.0)
    h_ref[...] = h
    tn_ref[...] = (
        jnp.dot(h, Wn_ref[...], preferred_element_type=jnp.float32)
        * dis_ref[...]
    )


_tc_b = pl.pallas_call(
    _tc_b_body,
    grid=(N // BN,),
    in_specs=[
        pl.BlockSpec((NC, BN, H), lambda i: (0, i, 0)),
        pl.BlockSpec((BN, H), lambda i: (i, 0)),
        pl.BlockSpec((BN, H), lambda i: (i, 0)),
        pl.BlockSpec((BN, 1), lambda i: (i, 0)),
        pl.BlockSpec((1, H), lambda i: (0, 0)),
        pl.BlockSpec((H, H), lambda i: (0, 0)),
    ],
    out_specs=[
        pl.BlockSpec((BN, H), lambda i: (i, 0)),
        pl.BlockSpec((BN, H), lambda i: (i, 0)),
    ],
    out_shape=[
        jax.ShapeDtypeStruct((N, H), jnp.float32),
        jax.ShapeDtypeStruct((N, H), jnp.float32),
    ],
)


def _tc_b3_body(acc_ref, t_ref, res_ref, dis_ref, b_ref, t4_ref):
    conv = (acc_ref[0] + acc_ref[1] + t_ref[...]) * dis_ref[...] + b_ref[...]
    h = jnp.maximum(conv + res_ref[...], 0.0)
    t4_ref[...] = h * dis_ref[...]


_tc_b3 = pl.pallas_call(
    _tc_b3_body,
    grid=(N // BN,),
    in_specs=[
        pl.BlockSpec((NC, BN, H), lambda i: (0, i, 0)),
        pl.BlockSpec((BN, H), lambda i: (i, 0)),
        pl.BlockSpec((BN, H), lambda i: (i, 0)),
        pl.BlockSpec((BN, 1), lambda i: (i, 0)),
        pl.BlockSpec((1, H), lambda i: (0, 0)),
    ],
    out_specs=[pl.BlockSpec((BN, H), lambda i: (i, 0))],
    out_shape=[jax.ShapeDtypeStruct((N, H), jnp.float32)],
)


def _tc_c_body(acc_ref, t_ref, dis_ref, W4_ref, b4_ref, out_ref):
    z = (acc_ref[0] + acc_ref[1] + t_ref[...]) * dis_ref[...]
    out_ref[...] = (
        jnp.dot(z, W4_ref[...], preferred_element_type=jnp.float32)
        + b4_ref[...]
    )


_tc_c = pl.pallas_call(
    _tc_c_body,
    grid=(N // BN,),
    in_specs=[
        pl.BlockSpec((NC, BN, H), lambda i: (0, i, 0)),
        pl.BlockSpec((BN, H), lambda i: (i, 0)),
        pl.BlockSpec((BN, 1), lambda i: (i, 0)),
        pl.BlockSpec((H, 1), lambda i: (0, 0)),
        pl.BlockSpec((1, 1), lambda i: (0, 0)),
    ],
    out_specs=[pl.BlockSpec((BN, 1), lambda i: (i, 0))],
    out_shape=[jax.ShapeDtypeStruct((N, 1), jnp.float32)],
)


def kernel(x, edge_index, edge_weight, W1, b1, W2, b2, W3, b3, W4, b4,
           Wres, bres):
    src = edge_index[0].astype(jnp.int32)
    dst = edge_index[1].astype(jnp.int32)
    w = edge_weight.astype(jnp.float32)
    e_in = src.shape[0]
    pad = E_CAP - e_in

    spread = (jnp.arange(pad, dtype=jnp.int32) * 97) % N
    src_p = jnp.concatenate([src, spread]).reshape(NW, NWIN, K)
    dummy = jnp.broadcast_to(
        ((jnp.arange(NBUF * K, dtype=jnp.int32) * 53) % N).reshape(1, NBUF, K),
        (NW, NBUF, K),
    )
    src3 = jnp.concatenate([src_p, dummy], axis=1)
    dst3 = jnp.concatenate([dst, spread]).reshape(NW, NWIN, K)
    w3 = jnp.concatenate([w, jnp.zeros((pad,), jnp.float32)]).reshape(
        NW, NWIN, K)
    zeros_n = jnp.zeros((NPAD,), jnp.float32)
    zeros_nh = jnp.zeros((NPAD, H), jnp.float32)

    _deg_kernel, _prop_kernel = _sc_kernels()
    deg_parts = _deg_kernel(dst3, w3, zeros_n)
    t1, xres, dis = _tc_a(deg_parts.reshape(NC, NPAD, 1), x, W1, Wres,
                          bres.reshape(1, H))
    acc1 = _prop_kernel(t1, src3, dst3, w3, zeros_nh)
    h1, t2 = _tc_b(acc1, t1, xres, dis, b1.reshape(1, H), W2)
    acc2 = _prop_kernel(t2, src3, dst3, w3, zeros_nh)
    h2, t3 = _tc_b(acc2, t2, h1, dis, b2.reshape(1, H), W3)
    acc3 = _prop_kernel(t3, src3, dst3, w3, zeros_nh)
    (t4,) = _tc_b3(acc3, t3, h2, dis, b3.reshape(1, H))
    acc4 = _prop_kernel(t4, src3, dst3, w3, zeros_nh)
    (out,) = _tc_c(acc4, t4, dis, W4, b4.reshape(1, 1))
    return out
